# Initial kernel scaffold; baseline (speedup 1.0000x reference)
import jax
import jax.numpy as jnp
from jax.experimental import pallas as pl


def kernel(features_batch, labels_batch):
    raise NotImplementedError("write your pallas kernel here")


# speedup vs baseline: 1.2112x; 1.2112x over previous
import jax
import jax.numpy as jnp
from jax import lax
from jax.experimental import pallas as pl
from jax.experimental.pallas import tpu as pltpu
from jax.experimental.pallas import tpu_sc as plsc

DD = 2.0
B = 4
F = 96
N = 224 * 224
NC, NS = 2, 16
NW = NC * NS
WPI = NW // B
CHUNK = N // WPI
NV = CHUNK // 16
FB = 8
NFB = F // FB
ROWS = F + 2
PWORDS = ROWS * 128


def _sc_body(feats_hbm, labels_hbm, out_hbm,
             addrbuf, buf0, buf1, sacc, qacc, cacc, partial,
             sem0, sem1):
    wid = lax.axis_index("s") * NC + lax.axis_index("c")
    img = wid // WPI
    base = (wid % WPI) * CHUNK
    zeros16 = jnp.zeros((16,), jnp.float32)
    ones16 = jnp.ones((16,), jnp.float32)
    iota16 = lax.iota(jnp.int32, 16)
    bufs = (buf0, buf1)
    sems = (sem0, sem1)

    def start(blk, b):
        pltpu.async_copy(
            feats_hbm.at[img, pl.ds(blk * FB, FB), pl.ds(base, CHUNK)],
            bufs[b], sems[b])

    def wait(blk, b):
        pltpu.make_async_copy(
            feats_hbm.at[img, pl.ds(blk * FB, FB), pl.ds(base, CHUNK)],
            bufs[b], sems[b]).wait()

    pltpu.sync_copy(labels_hbm.at[pl.ds(img * N + base, CHUNK)], addrbuf)
    for k in range(8):
        qacc[pl.ds(16 * k, 16)] = zeros16
        cacc[pl.ds(16 * k, 16)] = zeros16

    start(0, 0)
    start(1, 1)

    def addr_body(i, _):
        a = addrbuf[pl.ds(i * 16, 16)] * 16 + iota16
        addrbuf[pl.ds(i * 16, 16)] = a
        plsc.addupdate_scatter(cacc, [a], ones16)
        return 0

    lax.fori_loop(0, NV, addr_body, 0)

    def process(buf, blk):
        for f in range(FB):
            for k in range(8):
                sacc[pl.ds(16 * k, 16)] = zeros16

            def px_body(i, _):
                a = addrbuf[pl.ds(i * 16, 16)]
                v = buf[f, pl.ds(i * 16, 16)]
                plsc.addupdate_scatter(sacc, [a], v)
                plsc.addupdate_scatter(qacc, [a], v * v)
                return 0

            lax.fori_loop(0, NV, px_body, 0)
            row = blk * FB + f
            for k in range(8):
                partial[pl.ds(row * 128 + 16 * k, 16)] = \
                    sacc[pl.ds(16 * k, 16)]

    def pair_body(g2, _):
        g = g2 * 2
        wait(g, 0)
        process(buf0, g)

        @pl.when(g + 2 < NFB)
        def _():
            start(g + 2, 0)

        wait(g + 1, 1)
        process(buf1, g + 1)

        @pl.when(g + 3 < NFB)
        def _():
            start(g + 3, 1)

        return 0

    lax.fori_loop(0, NFB // 2, pair_body, 0)

    for k in range(8):
        partial[pl.ds(F * 128 + 16 * k, 16)] = qacc[pl.ds(16 * k, 16)]
        partial[pl.ds((F + 1) * 128 + 16 * k, 16)] = cacc[pl.ds(16 * k, 16)]

    pltpu.sync_copy(partial, out_hbm.at[pl.ds(wid * PWORDS, PWORDS)])


_sc_call = pl.kernel(
    _sc_body,
    out_type=jax.ShapeDtypeStruct((NW * PWORDS,), jnp.float32),
    mesh=plsc.VectorSubcoreMesh(
        core_axis_name="c", subcore_axis_name="s",
        num_cores=NC, num_subcores=NS),
    scratch_types=[
        pltpu.VMEM((CHUNK,), jnp.int32),
        pltpu.VMEM((FB, CHUNK), jnp.float32),
        pltpu.VMEM((FB, CHUNK), jnp.float32),
        pltpu.VMEM((128,), jnp.float32),
        pltpu.VMEM((128,), jnp.float32),
        pltpu.VMEM((128,), jnp.float32),
        pltpu.VMEM((PWORDS,), jnp.float32),
        pltpu.SemaphoreType.DMA,
        pltpu.SemaphoreType.DMA,
    ],
    compiler_params=pltpu.CompilerParams(needs_layout_passes=False),
)


def _epi_body(p_ref, o_ref):
    P = p_ref[...]
    col = lax.broadcasted_iota(jnp.int32, (WPI * 128, 8), 0)
    cls = lax.broadcasted_iota(jnp.int32, (WPI * 128, 8), 1)
    E = ((col // 16) % 8 == cls).astype(jnp.float32)
    R = jnp.dot(P, E, preferred_element_type=jnp.float32)

    ia = lax.broadcasted_iota(jnp.int32, (8, 8), 0)
    ib = lax.broadcasted_iota(jnp.int32, (8, 8), 1)
    eye = (ia == ib).astype(jnp.float32)
    upper = (ia < ib).astype(jnp.float32)
    clsrow = lax.broadcasted_iota(jnp.int32, (1, 8), 1)

    total = jnp.zeros((), jnp.float32)
    for img in range(B):
        S = lax.slice(R, (img * ROWS, 0), (img * ROWS + F, 8))
        Q = lax.slice(R, (img * ROWS + F, 0), (img * ROWS + F + 1, 8))
        C = lax.slice(R, (img * ROWS + F + 1, 0), (img * ROWS + ROWS, 8))
        size = jnp.maximum(C, 1.0)
        pres = (C > 0.5).astype(jnp.float32)
        s2 = jnp.sum(S * S, axis=0, keepdims=True)
        inner = jnp.where(clsrow == 0, Q, Q - s2 / size)
        var_loss = jnp.sum(pres * inner / size)

        M = jnp.where(clsrow == 0, 0.0, S / size)
        G = lax.dot_general(M, M, (((0,), (0,)), ((), ())),
                            preferred_element_type=jnp.float32)
        diag_col = jnp.sum(G * eye, axis=1, keepdims=True)
        n2_row = jnp.sum(M * M, axis=0, keepdims=True)
        d2 = jnp.maximum(diag_col + n2_row - 2.0 * G, 0.0)
        dist = jnp.sqrt(d2)
        pen = jnp.where(dist < 2.0 * DD, (2.0 * DD - dist) ** 2, 0.0)
        pres_col = jnp.sum(eye * pres, axis=1, keepdims=True)
        nc = jnp.sum(pres)
        denom = jnp.maximum(nc - 1.0, 1.0)
        dist_loss = jnp.sum(upper * pres_col * pres * pen) / denom
        tot = (var_loss + dist_loss) / jnp.maximum(nc, 1.0)
        total = total + jnp.where(nc <= 1.0, 0.0, tot)

    o_ref[...] = jnp.reshape(total / B, (1, 1))


_epi_call = pl.pallas_call(
    _epi_body,
    out_shape=jax.ShapeDtypeStruct((1, 1), jnp.float32),
    in_specs=[pl.BlockSpec((B * ROWS, WPI * 128), lambda: (0, 0))],
    out_specs=pl.BlockSpec((1, 1), lambda: (0, 0)),
)


@jax.jit
def kernel(features_batch, labels_batch):
    feats = features_batch.reshape(B, F, N)
    labels = labels_batch.reshape(B * N)
    partials = _sc_call(feats, labels)
    p2 = (partials.reshape(B, WPI, ROWS, 128)
          .transpose(0, 2, 1, 3)
          .reshape(B * ROWS, WPI * 128))
    loss = _epi_call(p2)
    return loss.reshape(1)


# speedup vs baseline: 2.2938x; 1.8939x over previous
import jax
import jax.numpy as jnp
from jax import lax
from jax.experimental import pallas as pl
from jax.experimental.pallas import tpu as pltpu
from jax.experimental.pallas import tpu_sc as plsc

DD = 2.0
B = 4
F = 96
N = 224 * 224
NC, NS = 2, 16
NW = NC * NS
WPI = NW // B
CHUNK = N // WPI
NV = CHUNK // 16
UNROLL = 2
FB = 8
NFB = F // FB
ROWS = F + 2
PWORDS = ROWS * 128
QOFF = F * 128
COFF = (F + 1) * 128


def _sc_body(feats_hbm, labels_hbm, out_hbm,
             addrbuf, buf0, buf1, partial, sem0, sem1):
    wid = lax.axis_index("s") * NC + lax.axis_index("c")
    img = wid // WPI
    base = (wid % WPI) * CHUNK
    zeros16 = jnp.zeros((16,), jnp.float32)
    ones16 = jnp.ones((16,), jnp.float32)
    iota16 = lax.iota(jnp.int32, 16)
    bufs = (buf0, buf1)
    sems = (sem0, sem1)

    def start(blk, b):
        pltpu.async_copy(
            feats_hbm.at[img, pl.ds(blk * FB, FB), pl.ds(base, CHUNK)],
            bufs[b], sems[b])

    def wait(blk, b):
        pltpu.make_async_copy(
            feats_hbm.at[img, pl.ds(blk * FB, FB), pl.ds(base, CHUNK)],
            bufs[b], sems[b]).wait()

    pltpu.sync_copy(labels_hbm.at[pl.ds(img * N + base, CHUNK)], addrbuf)
    start(0, 0)
    start(1, 1)

    def zero_body(i, _):
        for k in range(8):
            partial[pl.ds(i * 128 + 16 * k, 16)] = zeros16
        return 0

    lax.fori_loop(0, ROWS, zero_body, 0)

    def addr_body(i, _):
        for u in range(UNROLL):
            off = i * (16 * UNROLL) + u * 16
            a = addrbuf[pl.ds(off, 16)] * 16 + iota16
            addrbuf[pl.ds(off, 16)] = a
            plsc.addupdate_scatter(partial, [a + COFF], ones16)
        return 0

    lax.fori_loop(0, NV // UNROLL, addr_body, 0)

    def process(buf, blk):
        rowbase = blk * (FB * 128)

        def px_body(i, _):
            for u in range(UNROLL):
                off = i * (16 * UNROLL) + u * 16
                a = addrbuf[pl.ds(off, 16)]
                vs = [buf[f, pl.ds(off, 16)] for f in range(FB)]
                for f in range(FB):
                    plsc.addupdate_scatter(
                        partial, [a + (rowbase + f * 128)], vs[f])
                sq = [v * v for v in vs]
                while len(sq) > 1:
                    sq = [sq[2 * j] + sq[2 * j + 1]
                          for j in range(len(sq) // 2)]
                plsc.addupdate_scatter(partial, [a + QOFF], sq[0])
            return 0

        lax.fori_loop(0, NV // UNROLL, px_body, 0)

    def pair_body(g2, _):
        g = g2 * 2
        wait(g, 0)
        process(buf0, g)

        @pl.when(g + 2 < NFB)
        def _():
            start(g + 2, 0)

        wait(g + 1, 1)
        process(buf1, g + 1)

        @pl.when(g + 3 < NFB)
        def _():
            start(g + 3, 1)

        return 0

    lax.fori_loop(0, NFB // 2, pair_body, 0)

    pltpu.sync_copy(partial, out_hbm.at[pl.ds(wid * PWORDS, PWORDS)])


_sc_call = pl.kernel(
    _sc_body,
    out_type=jax.ShapeDtypeStruct((NW * PWORDS,), jnp.float32),
    mesh=plsc.VectorSubcoreMesh(
        core_axis_name="c", subcore_axis_name="s",
        num_cores=NC, num_subcores=NS),
    scratch_types=[
        pltpu.VMEM((CHUNK,), jnp.int32),
        pltpu.VMEM((FB, CHUNK), jnp.float32),
        pltpu.VMEM((FB, CHUNK), jnp.float32),
        pltpu.VMEM((PWORDS,), jnp.float32),
        pltpu.SemaphoreType.DMA,
        pltpu.SemaphoreType.DMA,
    ],
    compiler_params=pltpu.CompilerParams(needs_layout_passes=False),
)


def _epi_body(p_ref, o_ref):
    P = p_ref[...]
    col = lax.broadcasted_iota(jnp.int32, (WPI * 128, 8), 0)
    cls = lax.broadcasted_iota(jnp.int32, (WPI * 128, 8), 1)
    E = ((col // 16) % 8 == cls).astype(jnp.float32)
    R = jnp.dot(P, E, preferred_element_type=jnp.float32)

    ia = lax.broadcasted_iota(jnp.int32, (8, 8), 0)
    ib = lax.broadcasted_iota(jnp.int32, (8, 8), 1)
    eye = (ia == ib).astype(jnp.float32)
    upper = (ia < ib).astype(jnp.float32)
    clsrow = lax.broadcasted_iota(jnp.int32, (1, 8), 1)

    total = jnp.zeros((), jnp.float32)
    for img in range(B):
        S = lax.slice(R, (img * ROWS, 0), (img * ROWS + F, 8))
        Q = lax.slice(R, (img * ROWS + F, 0), (img * ROWS + F + 1, 8))
        C = lax.slice(R, (img * ROWS + F + 1, 0), (img * ROWS + ROWS, 8))
        size = jnp.maximum(C, 1.0)
        pres = (C > 0.5).astype(jnp.float32)
        s2 = jnp.sum(S * S, axis=0, keepdims=True)
        inner = jnp.where(clsrow == 0, Q, Q - s2 / size)
        var_loss = jnp.sum(pres * inner / size)

        M = jnp.where(clsrow == 0, 0.0, S / size)
        G = lax.dot_general(M, M, (((0,), (0,)), ((), ())),
                            preferred_element_type=jnp.float32)
        diag_col = jnp.sum(G * eye, axis=1, keepdims=True)
        n2_row = jnp.sum(M * M, axis=0, keepdims=True)
        d2 = jnp.maximum(diag_col + n2_row - 2.0 * G, 0.0)
        dist = jnp.sqrt(d2)
        pen = jnp.where(dist < 2.0 * DD, (2.0 * DD - dist) ** 2, 0.0)
        pres_col = jnp.sum(eye * pres, axis=1, keepdims=True)
        nc = jnp.sum(pres)
        denom = jnp.maximum(nc - 1.0, 1.0)
        dist_loss = jnp.sum(upper * pres_col * pres * pen) / denom
        tot = (var_loss + dist_loss) / jnp.maximum(nc, 1.0)
        total = total + jnp.where(nc <= 1.0, 0.0, tot)

    o_ref[...] = jnp.reshape(total / B, (1, 1))


_epi_call = pl.pallas_call(
    _epi_body,
    out_shape=jax.ShapeDtypeStruct((1, 1), jnp.float32),
    in_specs=[pl.BlockSpec((B * ROWS, WPI * 128), lambda: (0, 0))],
    out_specs=pl.BlockSpec((1, 1), lambda: (0, 0)),
)


@jax.jit
def kernel(features_batch, labels_batch):
    feats = features_batch.reshape(B, F, N)
    labels = labels_batch.reshape(B * N)
    partials = _sc_call(feats, labels)
    p2 = (partials.reshape(B, WPI, ROWS, 128)
          .transpose(0, 2, 1, 3)
          .reshape(B * ROWS, WPI * 128))
    loss = _epi_call(p2)
    return loss.reshape(1)


# speedup vs baseline: 3.6586x; 1.5950x over previous
import jax
import jax.numpy as jnp
from jax import lax
from jax.experimental import pallas as pl
from jax.experimental.pallas import tpu as pltpu
from jax.experimental.pallas import tpu_sc as plsc

DD = 2.0
B = 4
F = 96
H = 224
W = 224
NC, NS = 2, 16
WPI = 7
NWK = B * WPI
RB = H // WPI
NVC = W // 16
FB = 6
NFB = F // FB
ROWS = F + 2
PWORDS = ROWS * 128
QOFF = F * 128
COFF = (F + 1) * 128


def _sc_body(feats_hbm, labels_hbm, out_hbm,
             lblbuf, addrbuf, buf0, buf1, partial, sem0, sem1):
    wid = lax.axis_index("s") * NC + lax.axis_index("c")

    @pl.when(wid < NWK)
    def _():
        img = wid // WPI
        r0 = (wid % WPI) * RB
        zeros16 = jnp.zeros((16,), jnp.float32)
        ones16 = jnp.ones((16,), jnp.float32)
        iota16 = lax.iota(jnp.int32, 16)
        bufs = (buf0, buf1)
        sems = (sem0, sem1)

        def start(blk, b):
            pltpu.async_copy(
                feats_hbm.at[pl.ds(img * F + blk * FB, FB),
                             pl.ds(r0, RB), :],
                bufs[b], sems[b])

        def wait(blk, b):
            pltpu.make_async_copy(
                feats_hbm.at[pl.ds(img * F + blk * FB, FB),
                             pl.ds(r0, RB), :],
                bufs[b], sems[b]).wait()

        pltpu.sync_copy(labels_hbm.at[img, pl.ds(r0, RB), :], lblbuf)
        start(0, 0)
        start(1, 1)

        def zero_body(i, _):
            for k in range(8):
                partial[pl.ds(i * 128 + 16 * k, 16)] = zeros16
            return 0

        lax.fori_loop(0, ROWS, zero_body, 0)

        def addr_row(r, _):
            def addr_col(c, _c):
                a = lblbuf[r, pl.ds(c * 16, 16)] * 16 + iota16
                addrbuf[r, pl.ds(c * 16, 16)] = a
                plsc.addupdate_scatter(partial, [a + COFF], ones16)
                return 0
            lax.fori_loop(0, NVC, addr_col, 0)
            return 0

        lax.fori_loop(0, RB, addr_row, 0)

        def process(buf, blk):
            rowbase = blk * (FB * 128)
            sbases = [rowbase + f * 128 for f in range(FB)]

            def px_row(r, _):
                def px_col(c, _c):
                    a = addrbuf[r, pl.ds(c * 16, 16)]
                    vs = [buf[f, r, pl.ds(c * 16, 16)] for f in range(FB)]
                    for f in range(FB):
                        plsc.addupdate_scatter(
                            partial, [a + sbases[f]], vs[f])
                    sq = [v * v for v in vs]
                    while len(sq) > 1:
                        s2 = [sq[2 * j] + sq[2 * j + 1]
                              for j in range(len(sq) // 2)]
                        if len(sq) % 2:
                            s2[-1] = s2[-1] + sq[-1]
                        sq = s2
                    plsc.addupdate_scatter(partial, [a + QOFF], sq[0])
                    return 0
                lax.fori_loop(0, NVC, px_col, 0)
                return 0

            lax.fori_loop(0, RB, px_row, 0)

        def pair_body(g2, _):
            g = g2 * 2
            wait(g, 0)
            process(buf0, g)

            @pl.when(g + 2 < NFB)
            def _():
                start(g + 2, 0)

            wait(g + 1, 1)
            process(buf1, g + 1)

            @pl.when(g + 3 < NFB)
            def _():
                start(g + 3, 1)

            return 0

        lax.fori_loop(0, NFB // 2, pair_body, 0)

        pltpu.sync_copy(partial, out_hbm.at[pl.ds(wid * PWORDS, PWORDS)])


_sc_call = pl.kernel(
    _sc_body,
    out_type=jax.ShapeDtypeStruct((NWK * PWORDS,), jnp.float32),
    mesh=plsc.VectorSubcoreMesh(
        core_axis_name="c", subcore_axis_name="s",
        num_cores=NC, num_subcores=NS),
    scratch_types=[
        pltpu.VMEM((RB, W), jnp.int32),
        pltpu.VMEM((RB, W), jnp.int32),
        pltpu.VMEM((FB, RB, W), jnp.float32),
        pltpu.VMEM((FB, RB, W), jnp.float32),
        pltpu.VMEM((PWORDS,), jnp.float32),
        pltpu.SemaphoreType.DMA,
        pltpu.SemaphoreType.DMA,
    ],
    compiler_params=pltpu.CompilerParams(needs_layout_passes=False),
)


def _epi_body(p_ref, o_ref):
    P = p_ref[...]
    col = lax.broadcasted_iota(jnp.int32, (WPI * 128, 8), 0)
    cls = lax.broadcasted_iota(jnp.int32, (WPI * 128, 8), 1)
    E = ((col // 16) % 8 == cls).astype(jnp.float32)
    R = jnp.dot(P, E, preferred_element_type=jnp.float32)

    ia = lax.broadcasted_iota(jnp.int32, (8, 8), 0)
    ib = lax.broadcasted_iota(jnp.int32, (8, 8), 1)
    eye = (ia == ib).astype(jnp.float32)
    upper = (ia < ib).astype(jnp.float32)
    clsrow = lax.broadcasted_iota(jnp.int32, (1, 8), 1)

    total = jnp.zeros((), jnp.float32)
    for img in range(B):
        S = lax.slice(R, (img * ROWS, 0), (img * ROWS + F, 8))
        Q = lax.slice(R, (img * ROWS + F, 0), (img * ROWS + F + 1, 8))
        C = lax.slice(R, (img * ROWS + F + 1, 0), (img * ROWS + ROWS, 8))
        size = jnp.maximum(C, 1.0)
        pres = (C > 0.5).astype(jnp.float32)
        s2 = jnp.sum(S * S, axis=0, keepdims=True)
        inner = jnp.where(clsrow == 0, Q, Q - s2 / size)
        var_loss = jnp.sum(pres * inner / size)

        M = jnp.where(clsrow == 0, 0.0, S / size)
        G = lax.dot_general(M, M, (((0,), (0,)), ((), ())),
                            preferred_element_type=jnp.float32)
        diag_col = jnp.sum(G * eye, axis=1, keepdims=True)
        n2_row = jnp.sum(M * M, axis=0, keepdims=True)
        d2 = jnp.maximum(diag_col + n2_row - 2.0 * G, 0.0)
        dist = jnp.sqrt(d2)
        pen = jnp.where(dist < 2.0 * DD, (2.0 * DD - dist) ** 2, 0.0)
        pres_col = jnp.sum(eye * pres, axis=1, keepdims=True)
        nc = jnp.sum(pres)
        denom = jnp.maximum(nc - 1.0, 1.0)
        dist_loss = jnp.sum(upper * pres_col * pres * pen) / denom
        tot = (var_loss + dist_loss) / jnp.maximum(nc, 1.0)
        total = total + jnp.where(nc <= 1.0, 0.0, tot)

    o_ref[...] = jnp.reshape(total / B, (1, 1))


_epi_call = pl.pallas_call(
    _epi_body,
    out_shape=jax.ShapeDtypeStruct((1, 1), jnp.float32),
    in_specs=[pl.BlockSpec((B * ROWS, WPI * 128), lambda: (0, 0))],
    out_specs=pl.BlockSpec((1, 1), lambda: (0, 0)),
)


@jax.jit
def kernel(features_batch, labels_batch):
    feats = features_batch.reshape(B * F, H, W)
    partials = _sc_call(feats, labels_batch)
    p2 = (partials.reshape(B, WPI, ROWS, 128)
          .transpose(0, 2, 1, 3)
          .reshape(B * ROWS, WPI * 128))
    loss = _epi_call(p2)
    return loss.reshape(1)


# speedup vs baseline: 3.6733x; 1.0040x over previous
import jax
import jax.numpy as jnp
from jax import lax
from jax.experimental import pallas as pl
from jax.experimental.pallas import tpu as pltpu
from jax.experimental.pallas import tpu_sc as plsc

DD = 2.0
B = 4
F = 96
H = 224
W = 224
NC, NS = 2, 16
WPI = 7
NWK = B * WPI
RB = H // WPI
NVC = W // 16
FB = 6
NFB = F // FB
ROWS = F + 2
PWORDS = ROWS * 128
QOFF = F * 128
COFF = (F + 1) * 128


def _sc_body(feats_hbm, labels_hbm, out_hbm,
             lblbuf, addrbuf, buf0, buf1, partial, sem0, sem1):
    wid = lax.axis_index("s") * NC + lax.axis_index("c")

    @pl.when(wid < NWK)
    def _():
        img = wid // WPI
        r0 = (wid % WPI) * RB
        zeros16 = jnp.zeros((16,), jnp.float32)
        ones16 = jnp.ones((16,), jnp.float32)
        iota16 = lax.iota(jnp.int32, 16)
        bufs = (buf0, buf1)
        sems = (sem0, sem1)

        def start(blk, b):
            pltpu.async_copy(
                feats_hbm.at[pl.ds(img * F + blk * FB, FB),
                             pl.ds(r0, RB), :],
                bufs[b], sems[b])

        def wait(blk, b):
            pltpu.make_async_copy(
                feats_hbm.at[pl.ds(img * F + blk * FB, FB),
                             pl.ds(r0, RB), :],
                bufs[b], sems[b]).wait()

        pltpu.sync_copy(labels_hbm.at[img, pl.ds(r0, RB), :], lblbuf)
        start(0, 0)
        start(1, 1)

        def zero_body(i, _):
            for k in range(8):
                partial[pl.ds(i * 128 + 16 * k, 16)] = zeros16
            return 0

        lax.fori_loop(0, ROWS, zero_body, 0)

        def addr_row(r, _):
            for c in range(NVC):
                a = lblbuf[r, pl.ds(c * 16, 16)] * 16 + iota16
                addrbuf[r, pl.ds(c * 16, 16)] = a
                plsc.addupdate_scatter(partial, [a + COFF], ones16)
            return 0

        lax.fori_loop(0, RB, addr_row, 0)

        def process(buf, blk):
            rowbase = blk * (FB * 128)
            sbases = [rowbase + f * 128 for f in range(FB)]

            def px_row(r, _):
                for c in range(NVC):
                    a = addrbuf[r, pl.ds(c * 16, 16)]
                    vs = [buf[f, r, pl.ds(c * 16, 16)] for f in range(FB)]
                    for f in range(FB):
                        plsc.addupdate_scatter(
                            partial, [a + sbases[f]], vs[f])
                    sq = [v * v for v in vs]
                    while len(sq) > 1:
                        s2 = [sq[2 * j] + sq[2 * j + 1]
                              for j in range(len(sq) // 2)]
                        if len(sq) % 2:
                            s2[-1] = s2[-1] + sq[-1]
                        sq = s2
                    plsc.addupdate_scatter(partial, [a + QOFF], sq[0])
                return 0

            lax.fori_loop(0, RB, px_row, 0)

        def pair_body(g2, _):
            g = g2 * 2
            wait(g, 0)
            process(buf0, g)

            @pl.when(g + 2 < NFB)
            def _():
                start(g + 2, 0)

            wait(g + 1, 1)
            process(buf1, g + 1)

            @pl.when(g + 3 < NFB)
            def _():
                start(g + 3, 1)

            return 0

        lax.fori_loop(0, NFB // 2, pair_body, 0)

        pltpu.sync_copy(partial, out_hbm.at[pl.ds(wid * PWORDS, PWORDS)])


_sc_call = pl.kernel(
    _sc_body,
    out_type=jax.ShapeDtypeStruct((NWK * PWORDS,), jnp.float32),
    mesh=plsc.VectorSubcoreMesh(
        core_axis_name="c", subcore_axis_name="s",
        num_cores=NC, num_subcores=NS),
    scratch_types=[
        pltpu.VMEM((RB, W), jnp.int32),
        pltpu.VMEM((RB, W), jnp.int32),
        pltpu.VMEM((FB, RB, W), jnp.float32),
        pltpu.VMEM((FB, RB, W), jnp.float32),
        pltpu.VMEM((PWORDS,), jnp.float32),
        pltpu.SemaphoreType.DMA,
        pltpu.SemaphoreType.DMA,
    ],
    compiler_params=pltpu.CompilerParams(needs_layout_passes=False),
)


def _epi_body(p_ref, o_ref):
    P = p_ref[...]
    col = lax.broadcasted_iota(jnp.int32, (WPI * 128, 8), 0)
    cls = lax.broadcasted_iota(jnp.int32, (WPI * 128, 8), 1)
    E = ((col // 16) % 8 == cls).astype(jnp.float32)
    R = jnp.dot(P, E, preferred_element_type=jnp.float32)

    ia = lax.broadcasted_iota(jnp.int32, (8, 8), 0)
    ib = lax.broadcasted_iota(jnp.int32, (8, 8), 1)
    eye = (ia == ib).astype(jnp.float32)
    upper = (ia < ib).astype(jnp.float32)
    clsrow = lax.broadcasted_iota(jnp.int32, (1, 8), 1)

    total = jnp.zeros((), jnp.float32)
    for img in range(B):
        S = lax.slice(R, (img * ROWS, 0), (img * ROWS + F, 8))
        Q = lax.slice(R, (img * ROWS + F, 0), (img * ROWS + F + 1, 8))
        C = lax.slice(R, (img * ROWS + F + 1, 0), (img * ROWS + ROWS, 8))
        size = jnp.maximum(C, 1.0)
        pres = (C > 0.5).astype(jnp.float32)
        s2 = jnp.sum(S * S, axis=0, keepdims=True)
        inner = jnp.where(clsrow == 0, Q, Q - s2 / size)
        var_loss = jnp.sum(pres * inner / size)

        M = jnp.where(clsrow == 0, 0.0, S / size)
        G = lax.dot_general(M, M, (((0,), (0,)), ((), ())),
                            preferred_element_type=jnp.float32)
        diag_col = jnp.sum(G * eye, axis=1, keepdims=True)
        n2_row = jnp.sum(M * M, axis=0, keepdims=True)
        d2 = jnp.maximum(diag_col + n2_row - 2.0 * G, 0.0)
        dist = jnp.sqrt(d2)
        pen = jnp.where(dist < 2.0 * DD, (2.0 * DD - dist) ** 2, 0.0)
        pres_col = jnp.sum(eye * pres, axis=1, keepdims=True)
        nc = jnp.sum(pres)
        denom = jnp.maximum(nc - 1.0, 1.0)
        dist_loss = jnp.sum(upper * pres_col * pres * pen) / denom
        tot = (var_loss + dist_loss) / jnp.maximum(nc, 1.0)
        total = total + jnp.where(nc <= 1.0, 0.0, tot)

    o_ref[...] = jnp.reshape(total / B, (1, 1))


_epi_call = pl.pallas_call(
    _epi_body,
    out_shape=jax.ShapeDtypeStruct((1, 1), jnp.float32),
    in_specs=[pl.BlockSpec((B * ROWS, WPI * 128), lambda: (0, 0))],
    out_specs=pl.BlockSpec((1, 1), lambda: (0, 0)),
)


@jax.jit
def kernel(features_batch, labels_batch):
    feats = features_batch.reshape(B * F, H, W)
    partials = _sc_call(feats, labels_batch)
    p2 = (partials.reshape(B, WPI, ROWS, 128)
          .transpose(0, 2, 1, 3)
          .reshape(B * ROWS, WPI * 128))
    loss = _epi_call(p2)
    return loss.reshape(1)


# speedup vs baseline: 5.6210x; 1.5302x over previous
import jax
import jax.numpy as jnp
from jax import lax
from jax.experimental import pallas as pl
from jax.experimental.pallas import tpu as pltpu
from jax.experimental.pallas import tpu_sc as plsc

DD = 2.0
B = 4
F = 96
H = 224
W = 224
NC, NS = 2, 16
WPI = 7
NWK = B * WPI
RB = H // WPI
NVC = W // 16
FSC = 48
FTC = F - FSC
FB = 6
NFB = FSC // FB
ROWS = FSC + 2
PWORDS = ROWS * 128
QOFF = FSC * 128
COFF = (FSC + 1) * 128
TSTRIPE = 32
NTS = H // TSTRIPE


def _sc_body(feats_hbm, labels_hbm, out_hbm,
             lblbuf, addrbuf, buf0, buf1, partial, sem0, sem1):
    wid = lax.axis_index("s") * NC + lax.axis_index("c")

    @pl.when(wid < NWK)
    def _():
        img = wid // WPI
        r0 = (wid % WPI) * RB
        zeros16 = jnp.zeros((16,), jnp.float32)
        ones16 = jnp.ones((16,), jnp.float32)
        iota16 = lax.iota(jnp.int32, 16)
        bufs = (buf0, buf1)
        sems = (sem0, sem1)

        def start(blk, b):
            pltpu.async_copy(
                feats_hbm.at[pl.ds(img * F + blk * FB, FB),
                             pl.ds(r0, RB), :],
                bufs[b], sems[b])

        def wait(blk, b):
            pltpu.make_async_copy(
                feats_hbm.at[pl.ds(img * F + blk * FB, FB),
                             pl.ds(r0, RB), :],
                bufs[b], sems[b]).wait()

        pltpu.sync_copy(labels_hbm.at[img, pl.ds(r0, RB), :], lblbuf)
        start(0, 0)
        start(1, 1)

        def zero_body(i, _):
            for k in range(8):
                partial[pl.ds(i * 128 + 16 * k, 16)] = zeros16
            return 0

        lax.fori_loop(0, ROWS, zero_body, 0)

        def addr_row(r, _):
            for c in range(NVC):
                a = lblbuf[r, pl.ds(c * 16, 16)] * 16 + iota16
                addrbuf[r, pl.ds(c * 16, 16)] = a
                plsc.addupdate_scatter(partial, [a + COFF], ones16)
            return 0

        lax.fori_loop(0, RB, addr_row, 0)

        def process(buf, blk):
            rowbase = blk * (FB * 128)
            sbases = [rowbase + f * 128 for f in range(FB)]

            def px_row(r, _):
                for c in range(NVC):
                    a = addrbuf[r, pl.ds(c * 16, 16)]
                    vs = [buf[f, r, pl.ds(c * 16, 16)] for f in range(FB)]
                    for f in range(FB):
                        plsc.addupdate_scatter(
                            partial, [a + sbases[f]], vs[f])
                    sq = [v * v for v in vs]
                    while len(sq) > 1:
                        s2 = [sq[2 * j] + sq[2 * j + 1]
                              for j in range(len(sq) // 2)]
                        if len(sq) % 2:
                            s2[-1] = s2[-1] + sq[-1]
                        sq = s2
                    plsc.addupdate_scatter(partial, [a + QOFF], sq[0])
                return 0

            lax.fori_loop(0, RB, px_row, 0)

        def pair_body(g2, _):
            g = g2 * 2
            wait(g, 0)
            process(buf0, g)

            @pl.when(g + 2 < NFB)
            def _():
                start(g + 2, 0)

            wait(g + 1, 1)
            process(buf1, g + 1)

            @pl.when(g + 3 < NFB)
            def _():
                start(g + 3, 1)

            return 0

        lax.fori_loop(0, NFB // 2, pair_body, 0)

        pltpu.sync_copy(partial, out_hbm.at[pl.ds(wid * PWORDS, PWORDS)])


_sc_call = pl.kernel(
    _sc_body,
    out_type=jax.ShapeDtypeStruct((NWK * PWORDS,), jnp.float32),
    mesh=plsc.VectorSubcoreMesh(
        core_axis_name="c", subcore_axis_name="s",
        num_cores=NC, num_subcores=NS),
    scratch_types=[
        pltpu.VMEM((RB, W), jnp.int32),
        pltpu.VMEM((RB, W), jnp.int32),
        pltpu.VMEM((FB, RB, W), jnp.float32),
        pltpu.VMEM((FB, RB, W), jnp.float32),
        pltpu.VMEM((PWORDS,), jnp.float32),
        pltpu.SemaphoreType.DMA,
        pltpu.SemaphoreType.DMA,
    ],
    compiler_params=pltpu.CompilerParams(needs_layout_passes=False),
)


def _tc_stats_body(f_ref, l_ref, s_ref, q_ref):
    t = pl.program_id(1)

    @pl.when(t == 0)
    def _():
        s_ref[...] = jnp.zeros_like(s_ref)
        q_ref[...] = jnp.zeros_like(q_ref)

    feats = f_ref[...]
    lbl = l_ref[0]
    qpx = jnp.sum(feats * feats, axis=0)
    svals = []
    qvals = []
    for c in range(8):
        mf = (lbl == c).astype(jnp.float32)
        if c > 0:
            svals.append(jnp.sum(feats * mf[None], axis=(1, 2)))
        qvals.append(jnp.sum(qpx * mf))
    s_ref[0, 1:8, :] += jnp.stack(svals, axis=0)
    q_ref[0, 0, :] += jnp.stack(qvals)


_tc_stats_call = pl.pallas_call(
    _tc_stats_body,
    grid=(B, NTS),
    in_specs=[
        pl.BlockSpec((FTC, TSTRIPE, W),
                     lambda i, t: (2 * i + 1, t, 0)),
        pl.BlockSpec((1, TSTRIPE, W), lambda i, t: (i, t, 0)),
    ],
    out_specs=[
        pl.BlockSpec((1, 8, FTC), lambda i, t: (i, 0, 0)),
        pl.BlockSpec((1, 1, 8), lambda i, t: (i, 0, 0)),
    ],
    out_shape=[
        jax.ShapeDtypeStruct((B, 8, FTC), jnp.float32),
        jax.ShapeDtypeStruct((B, 1, 8), jnp.float32),
    ],
)


def _epi_body(p_ref, st_ref, qt_ref, o_ref):
    P = p_ref[...]
    col = lax.broadcasted_iota(jnp.int32, (WPI * 128, 8), 0)
    cls = lax.broadcasted_iota(jnp.int32, (WPI * 128, 8), 1)
    E = ((col // 16) % 8 == cls).astype(jnp.float32)
    R = jnp.dot(P, E, preferred_element_type=jnp.float32)

    ia = lax.broadcasted_iota(jnp.int32, (8, 8), 0)
    ib = lax.broadcasted_iota(jnp.int32, (8, 8), 1)
    eye = (ia == ib).astype(jnp.float32)
    upper = (ia < ib).astype(jnp.float32)
    clsrow = lax.broadcasted_iota(jnp.int32, (1, 8), 1)
    fa = lax.broadcasted_iota(jnp.int32, (FTC, FTC), 0)
    fb = lax.broadcasted_iota(jnp.int32, (FTC, FTC), 1)
    eyef = (fa == fb).astype(jnp.float32)

    total = jnp.zeros((), jnp.float32)
    for img in range(B):
        S_sc = lax.slice(R, (img * ROWS, 0), (img * ROWS + FSC, 8))
        Q = (lax.slice(R, (img * ROWS + FSC, 0), (img * ROWS + FSC + 1, 8))
             + qt_ref[img])
        C = lax.slice(R, (img * ROWS + FSC + 1, 0), (img * ROWS + ROWS, 8))
        S_tc = lax.dot_general(eyef, st_ref[img], (((1,), (1,)), ((), ())),
                               preferred_element_type=jnp.float32)
        S = jnp.concatenate([S_sc, S_tc], axis=0)
        size = jnp.maximum(C, 1.0)
        pres = (C > 0.5).astype(jnp.float32)
        s2 = jnp.sum(S * S, axis=0, keepdims=True)
        inner = jnp.where(clsrow == 0, Q, Q - s2 / size)
        var_loss = jnp.sum(pres * inner / size)

        M = jnp.where(clsrow == 0, 0.0, S / size)
        G = lax.dot_general(M, M, (((0,), (0,)), ((), ())),
                            preferred_element_type=jnp.float32)
        diag_col = jnp.sum(G * eye, axis=1, keepdims=True)
        n2_row = jnp.sum(M * M, axis=0, keepdims=True)
        d2 = jnp.maximum(diag_col + n2_row - 2.0 * G, 0.0)
        dist = jnp.sqrt(d2)
        pen = jnp.where(dist < 2.0 * DD, (2.0 * DD - dist) ** 2, 0.0)
        pres_col = jnp.sum(eye * pres, axis=1, keepdims=True)
        nc = jnp.sum(pres)
        denom = jnp.maximum(nc - 1.0, 1.0)
        dist_loss = jnp.sum(upper * pres_col * pres * pen) / denom
        tot = (var_loss + dist_loss) / jnp.maximum(nc, 1.0)
        total = total + jnp.where(nc <= 1.0, 0.0, tot)

    o_ref[...] = jnp.reshape(total / B, (1, 1))


_epi_call = pl.pallas_call(
    _epi_body,
    out_shape=jax.ShapeDtypeStruct((1, 1), jnp.float32),
    in_specs=[
        pl.BlockSpec((B * ROWS, WPI * 128), lambda: (0, 0)),
        pl.BlockSpec((B, 8, FTC), lambda: (0, 0, 0)),
        pl.BlockSpec((B, 1, 8), lambda: (0, 0, 0)),
    ],
    out_specs=pl.BlockSpec((1, 1), lambda: (0, 0)),
)


@jax.jit
def kernel(features_batch, labels_batch):
    feats = features_batch.reshape(B * F, H, W)
    partials = _sc_call(feats, labels_batch)
    s_tc, q_tc = _tc_stats_call(feats, labels_batch)
    p2 = (partials.reshape(B, WPI, ROWS, 128)
          .transpose(0, 2, 1, 3)
          .reshape(B * ROWS, WPI * 128))
    loss = _epi_call(p2, s_tc, q_tc)
    return loss.reshape(1)


# speedup vs baseline: 5.6813x; 1.0107x over previous
import jax
import jax.numpy as jnp
from jax import lax
from jax.experimental import pallas as pl
from jax.experimental.pallas import tpu as pltpu
from jax.experimental.pallas import tpu_sc as plsc

DD = 2.0
B = 4
F = 96
H = 224
W = 224
NC, NS = 2, 16
WPI = 7
NWK = B * WPI
RB = H // WPI
NVC = W // 16
FSC = 48
FTC = F - FSC
FB = 6
NFB = FSC // FB
ROWS = FSC + 2
PWORDS = ROWS * 128
QOFF = FSC * 128
COFF = (FSC + 1) * 128
TSTRIPE = 32
NTS = H // TSTRIPE


def _sc_body(feats_hbm, labels_hbm, out_hbm,
             lblbuf, addrbuf, buf0, buf1, partial, sem0, sem1):
    wid = lax.axis_index("s") * NC + lax.axis_index("c")

    @pl.when(wid < NWK)
    def _():
        img = wid // WPI
        r0 = (wid % WPI) * RB
        zeros16 = jnp.zeros((16,), jnp.float32)
        ones16 = jnp.ones((16,), jnp.float32)
        iota16 = lax.iota(jnp.int32, 16)
        bufs = (buf0, buf1)
        sems = (sem0, sem1)

        def start(blk, b):
            pltpu.async_copy(
                feats_hbm.at[pl.ds(img * F + blk * FB, FB),
                             pl.ds(r0, RB), :],
                bufs[b], sems[b])

        def wait(blk, b):
            pltpu.make_async_copy(
                feats_hbm.at[pl.ds(img * F + blk * FB, FB),
                             pl.ds(r0, RB), :],
                bufs[b], sems[b]).wait()

        pltpu.sync_copy(labels_hbm.at[img, pl.ds(r0, RB), :], lblbuf)
        start(0, 0)
        start(1, 1)

        def zero_body(i, _):
            for k in range(8):
                partial[pl.ds(i * 128 + 16 * k, 16)] = zeros16
            return 0

        lax.fori_loop(0, ROWS, zero_body, 0)

        def addr_row(r, _):
            for c in range(NVC):
                a = lblbuf[r, pl.ds(c * 16, 16)] * 16 + iota16
                addrbuf[r, pl.ds(c * 16, 16)] = a
                plsc.addupdate_scatter(partial, [a + COFF], ones16)
            return 0

        lax.fori_loop(0, RB, addr_row, 0)

        def process(buf, blk):
            rowbase = blk * (FB * 128)
            sbases = [rowbase + f * 128 for f in range(FB)]

            def px_row(r, _):
                for c in range(NVC):
                    a = addrbuf[r, pl.ds(c * 16, 16)]
                    vs = [buf[f, r, pl.ds(c * 16, 16)] for f in range(FB)]
                    for f in range(FB):
                        plsc.addupdate_scatter(
                            partial, [a + sbases[f]], vs[f])
                    sq = [v * v for v in vs]
                    while len(sq) > 1:
                        s2 = [sq[2 * j] + sq[2 * j + 1]
                              for j in range(len(sq) // 2)]
                        if len(sq) % 2:
                            s2[-1] = s2[-1] + sq[-1]
                        sq = s2
                    plsc.addupdate_scatter(partial, [a + QOFF], sq[0])
                return 0

            lax.fori_loop(0, RB, px_row, 0)

        def pair_body(g2, _):
            g = g2 * 2
            wait(g, 0)
            process(buf0, g)

            @pl.when(g + 2 < NFB)
            def _():
                start(g + 2, 0)

            wait(g + 1, 1)
            process(buf1, g + 1)

            @pl.when(g + 3 < NFB)
            def _():
                start(g + 3, 1)

            return 0

        lax.fori_loop(0, NFB // 2, pair_body, 0)

        pltpu.sync_copy(partial, out_hbm.at[pl.ds(wid * PWORDS, PWORDS)])


_sc_call = pl.kernel(
    _sc_body,
    out_type=jax.ShapeDtypeStruct((NWK * PWORDS,), jnp.float32),
    mesh=plsc.VectorSubcoreMesh(
        core_axis_name="c", subcore_axis_name="s",
        num_cores=NC, num_subcores=NS),
    scratch_types=[
        pltpu.VMEM((RB, W), jnp.int32),
        pltpu.VMEM((RB, W), jnp.int32),
        pltpu.VMEM((FB, RB, W), jnp.float32),
        pltpu.VMEM((FB, RB, W), jnp.float32),
        pltpu.VMEM((PWORDS,), jnp.float32),
        pltpu.SemaphoreType.DMA,
        pltpu.SemaphoreType.DMA,
    ],
    compiler_params=pltpu.CompilerParams(needs_layout_passes=False),
)


def _tc_stats_body(f_ref, l_ref, s_ref, q_ref):
    t = pl.program_id(1)

    @pl.when(t == 0)
    def _():
        s_ref[...] = jnp.zeros_like(s_ref)
        q_ref[...] = jnp.zeros_like(q_ref)

    feats = f_ref[...]
    lbl = l_ref[0]
    qpx = jnp.sum(feats * feats, axis=0)
    svals = []
    qvals = []
    for c in range(8):
        mf = (lbl == c).astype(jnp.float32)
        if c > 0:
            svals.append(jnp.sum(feats * mf[None], axis=(1, 2)))
        qvals.append(jnp.sum(qpx * mf))
    s_ref[0, 1:8, :] += jnp.stack(svals, axis=0)
    q_ref[0, 0, :] += jnp.stack(qvals)


_tc_stats_call = pl.pallas_call(
    _tc_stats_body,
    grid=(B, NTS),
    in_specs=[
        pl.BlockSpec((FTC, TSTRIPE, W),
                     lambda i, t: (2 * i + 1, t, 0)),
        pl.BlockSpec((1, TSTRIPE, W), lambda i, t: (i, t, 0)),
    ],
    out_specs=[
        pl.BlockSpec((1, 8, FTC), lambda i, t: (i, 0, 0)),
        pl.BlockSpec((1, 1, 8), lambda i, t: (i, 0, 0)),
    ],
    out_shape=[
        jax.ShapeDtypeStruct((B, 8, FTC), jnp.float32),
        jax.ShapeDtypeStruct((B, 1, 8), jnp.float32),
    ],
)


def _epi_body(p_ref, st_ref, qt_ref, o_ref):
    col = lax.broadcasted_iota(jnp.int32, (128, 8), 0)
    cls = lax.broadcasted_iota(jnp.int32, (128, 8), 1)
    E = ((col // 16) == cls).astype(jnp.float32)
    rs = []
    for img in range(B):
        Pimg = p_ref[img * WPI]
        for k in range(1, WPI):
            Pimg = Pimg + p_ref[img * WPI + k]
        rs.append(jnp.dot(Pimg, E, preferred_element_type=jnp.float32))
    R = jnp.concatenate(rs, axis=0)

    ia = lax.broadcasted_iota(jnp.int32, (8, 8), 0)
    ib = lax.broadcasted_iota(jnp.int32, (8, 8), 1)
    eye = (ia == ib).astype(jnp.float32)
    upper = (ia < ib).astype(jnp.float32)
    clsrow = lax.broadcasted_iota(jnp.int32, (1, 8), 1)
    fa = lax.broadcasted_iota(jnp.int32, (FTC, FTC), 0)
    fb = lax.broadcasted_iota(jnp.int32, (FTC, FTC), 1)
    eyef = (fa == fb).astype(jnp.float32)

    total = jnp.zeros((), jnp.float32)
    for img in range(B):
        S_sc = lax.slice(R, (img * ROWS, 0), (img * ROWS + FSC, 8))
        Q = (lax.slice(R, (img * ROWS + FSC, 0), (img * ROWS + FSC + 1, 8))
             + qt_ref[img])
        C = lax.slice(R, (img * ROWS + FSC + 1, 0), (img * ROWS + ROWS, 8))
        S_tc = lax.dot_general(eyef, st_ref[img], (((1,), (1,)), ((), ())),
                               preferred_element_type=jnp.float32)
        S = jnp.concatenate([S_sc, S_tc], axis=0)
        size = jnp.maximum(C, 1.0)
        pres = (C > 0.5).astype(jnp.float32)
        s2 = jnp.sum(S * S, axis=0, keepdims=True)
        inner = jnp.where(clsrow == 0, Q, Q - s2 / size)
        var_loss = jnp.sum(pres * inner / size)

        M = jnp.where(clsrow == 0, 0.0, S / size)
        G = lax.dot_general(M, M, (((0,), (0,)), ((), ())),
                            preferred_element_type=jnp.float32)
        diag_col = jnp.sum(G * eye, axis=1, keepdims=True)
        n2_row = jnp.sum(M * M, axis=0, keepdims=True)
        d2 = jnp.maximum(diag_col + n2_row - 2.0 * G, 0.0)
        dist = jnp.sqrt(d2)
        pen = jnp.where(dist < 2.0 * DD, (2.0 * DD - dist) ** 2, 0.0)
        pres_col = jnp.sum(eye * pres, axis=1, keepdims=True)
        nc = jnp.sum(pres)
        denom = jnp.maximum(nc - 1.0, 1.0)
        dist_loss = jnp.sum(upper * pres_col * pres * pen) / denom
        tot = (var_loss + dist_loss) / jnp.maximum(nc, 1.0)
        total = total + jnp.where(nc <= 1.0, 0.0, tot)

    o_ref[...] = jnp.reshape(total / B, (1, 1))


_epi_call = pl.pallas_call(
    _epi_body,
    out_shape=jax.ShapeDtypeStruct((1, 1), jnp.float32),
    in_specs=[
        pl.BlockSpec((NWK, ROWS, 128), lambda: (0, 0, 0)),
        pl.BlockSpec((B, 8, FTC), lambda: (0, 0, 0)),
        pl.BlockSpec((B, 1, 8), lambda: (0, 0, 0)),
    ],
    out_specs=pl.BlockSpec((1, 1), lambda: (0, 0)),
)


@jax.jit
def kernel(features_batch, labels_batch):
    feats = features_batch.reshape(B * F, H, W)
    partials = _sc_call(feats, labels_batch)
    s_tc, q_tc = _tc_stats_call(feats, labels_batch)
    p3 = partials.reshape(NWK, ROWS, 128)
    loss = _epi_call(p3, s_tc, q_tc)
    return loss.reshape(1)
